# baseline (speedup 1.0000x reference)
import functools

import jax
import jax.numpy as jnp
import numpy as np
from jax import lax
from jax.experimental import pallas as pl
from jax.experimental.pallas import tpu as pltpu
from jax.experimental.pallas import tpu_sc as plsc

D = 768
LANES = 16
PAIRS = D // (2 * LANES)
NW = 32
NS = 16
B_PER_W = 4
CHUNK = 16
NBUF = 4
OBUF = 2

_PERM = np.arange(D).reshape(PAIRS, 2, LANES).transpose(0, 2, 1).reshape(-1)


def _sc_body(gidx_hbm, ct_hbm, pe_hbm, out_hbm,
             idx_all, pe_v, rows, obufs, pe_sh, gsem, wsem):
    n_rows = out_hbm.shape[0]
    p_per_b = pe_hbm.shape[0]
    rows_per_w = n_rows // NW
    pcs = p_per_b // CHUNK
    nsteps = pcs * B_PER_W
    sid = lax.axis_index("s")
    wid = sid * 2 + lax.axis_index("c")
    w0 = wid * rows_per_w

    pps = p_per_b // NS
    for j in range(pps // CHUNK):
        r0 = sid * pps + j * CHUNK
        pltpu.sync_copy(pe_hbm.at[pl.ds(r0, CHUNK)], rows[0])
        pltpu.sync_copy(rows[0], pe_sh.at[pl.ds(r0, CHUNK)])
    plsc.subcore_barrier()

    def idx_off(s):
        pc = s // B_PER_W
        bi = lax.rem(s, B_PER_W)
        return bi * p_per_b + pc * CHUNK

    pltpu.sync_copy(gidx_hbm.at[pl.ds(w0, rows_per_w)], idx_all)
    pltpu.async_copy(ct_hbm.at[idx_all.at[pl.ds(idx_off(0), CHUNK)]], rows[0], gsem[0])
    pltpu.async_copy(ct_hbm.at[idx_all.at[pl.ds(idx_off(1), CHUNK)]], rows[1], gsem[1])

    def outer(i, carry):
        for k in range(NBUF):
            s = i * NBUF + k
            pc = s // B_PER_W
            base = w0 + lax.rem(s, B_PER_W) * p_per_b + pc * CHUNK
            rx, gs = rows[k], gsem[k]
            ob, ws = obufs[k % OBUF], wsem[k % OBUF]
            k2 = (k + 2) % NBUF
            @pl.when(s + 2 < nsteps)
            def _():
                pltpu.async_copy(
                    ct_hbm.at[idx_all.at[pl.ds(idx_off(s + 2), CHUNK)]],
                    rows[k2], gsem[k2])

            if k == 0:
                pltpu.sync_copy(pe_sh.at[pl.ds(pc * CHUNK, CHUNK)], pe_v)

            pltpu.make_async_copy(ct_hbm.at[idx_all.at[pl.ds(0, CHUNK)]],
                                  rx, gs).wait()

            @pl.when(s >= OBUF)
            def _():
                pltpu.make_async_copy(ob, out_hbm.at[pl.ds(base, CHUNK)],
                                      ws).wait()

            @plsc.parallel_loop(0, PAIRS, step=1, unroll=3)
            def _(c):
                ci = pl.multiple_of(c * LANES, LANES)
                co = pl.multiple_of(c * 32, 32)
                for r in range(CHUNK):
                    uc = rx[r, pl.ds(ci, LANES)]
                    up = pe_v[r, pl.ds(ci, LANES)]
                    hi = jnp.int32(-65536)
                    a = (lax.bitcast_convert_type(uc << 16, jnp.float32)
                         + lax.bitcast_convert_type(up << 16, jnp.float32))
                    b = (lax.bitcast_convert_type(uc & hi, jnp.float32)
                         + lax.bitcast_convert_type(up & hi, jnp.float32))
                    ob[r, pl.ds(co, LANES)] = a
                    ob[r, pl.ds(co + LANES, LANES)] = b

            pltpu.async_copy(ob, out_hbm.at[pl.ds(base, CHUNK)], ws)
        return carry

    lax.fori_loop(0, nsteps // NBUF, outer, 0, unroll=False)
    for k in ((nsteps - 2) % OBUF, (nsteps - 1) % OBUF):
        pltpu.make_async_copy(obufs[k], out_hbm.at[pl.ds(w0, CHUNK)],
                              wsem[k]).wait()


@jax.jit
def _embed(gidx, ct, pe):
    n_rows = gidx.shape[0]
    mesh = plsc.VectorSubcoreMesh(core_axis_name="c", subcore_axis_name="s")
    f = functools.partial(
        pl.kernel,
        out_type=jax.ShapeDtypeStruct((n_rows, D), jnp.float32),
        mesh=mesh,
        scratch_types=[
            pltpu.VMEM((n_rows // NW,), jnp.int32),
            pltpu.VMEM((CHUNK, D // 2), jnp.int32),
            [pltpu.VMEM((CHUNK, D // 2), jnp.int32)] * NBUF,
            [pltpu.VMEM((CHUNK, D), jnp.float32)] * OBUF,
            pltpu.VMEM_SHARED((512, D // 2), jnp.int32),
            [pltpu.SemaphoreType.DMA] * NBUF,
            [pltpu.SemaphoreType.DMA] * OBUF,
        ],
    )(_sc_body)
    return f(gidx, ct, pe)


def kernel(tokens, actions, tok_embed0, tok_embed1, tok_embed2, action_embed,
           level_embed, pos_embed):
    B, T, _ = tokens.shape
    num_codes = tok_embed0.shape[0]
    ct = jnp.concatenate(
        [
            tok_embed0 + level_embed[0],
            tok_embed1 + level_embed[1],
            tok_embed2 + level_embed[2],
            action_embed + level_embed[3],
        ],
        axis=0,
    )
    perm = jnp.asarray(_PERM)
    ct = ct[:, perm].astype(jnp.bfloat16)
    ct = lax.bitcast_convert_type(ct.reshape(ct.shape[0], D // 2, 2), jnp.int32)
    pe = pos_embed[: T * 4][:, perm].astype(jnp.bfloat16)
    pe = lax.bitcast_convert_type(pe.reshape(T * 4, D // 2, 2), jnp.int32)
    gidx = jnp.stack(
        [
            tokens[..., 0],
            tokens[..., 1] + num_codes,
            tokens[..., 2] + 2 * num_codes,
            actions + 3 * num_codes,
        ],
        axis=-1,
    ).reshape(-1)
    out = _embed(gidx, ct, pe)
    return out.reshape(B, T * 4, D)


# speedup vs baseline: 1.0565x; 1.0565x over previous
import functools

import jax
import jax.numpy as jnp
import numpy as np
from jax import lax
from jax.experimental import pallas as pl
from jax.experimental.pallas import tpu as pltpu
from jax.experimental.pallas import tpu_sc as plsc

D = 768
LANES = 16
PAIRS = D // (2 * LANES)
NW = 32
NS = 16
B_PER_W = 4
CHUNK = 32
NBUF = 4
OBUF = 2

_PERM = np.arange(D).reshape(PAIRS, 2, LANES).transpose(0, 2, 1).reshape(-1)


def _sc_body(gidx_hbm, ct_hbm, pe_hbm, out_hbm,
             idx_all, pe_v, rows, obufs, pe_sh, gsem, wsem):
    n_rows = out_hbm.shape[0]
    p_per_b = pe_hbm.shape[0]
    rows_per_w = n_rows // NW
    pcs = p_per_b // CHUNK
    nsteps = pcs * B_PER_W
    sid = lax.axis_index("s")
    wid = sid * 2 + lax.axis_index("c")
    w0 = wid * rows_per_w

    pps = p_per_b // NS
    for j in range(pps // CHUNK):
        r0 = sid * pps + j * CHUNK
        pltpu.sync_copy(pe_hbm.at[pl.ds(r0, CHUNK)], rows[0])
        pltpu.sync_copy(rows[0], pe_sh.at[pl.ds(r0, CHUNK)])
    plsc.subcore_barrier()

    def idx_off(s):
        pc = s // B_PER_W
        bi = lax.rem(s, B_PER_W)
        return bi * p_per_b + pc * CHUNK

    pltpu.sync_copy(gidx_hbm.at[pl.ds(w0, rows_per_w)], idx_all)
    pltpu.async_copy(ct_hbm.at[idx_all.at[pl.ds(idx_off(0), CHUNK)]], rows[0], gsem[0])
    pltpu.async_copy(ct_hbm.at[idx_all.at[pl.ds(idx_off(1), CHUNK)]], rows[1], gsem[1])

    def outer(i, carry):
        for k in range(NBUF):
            s = i * NBUF + k
            pc = s // B_PER_W
            base = w0 + lax.rem(s, B_PER_W) * p_per_b + pc * CHUNK
            rx, gs = rows[k], gsem[k]
            ob, ws = obufs[k % OBUF], wsem[k % OBUF]
            k2 = (k + 2) % NBUF
            @pl.when(s + 2 < nsteps)
            def _():
                pltpu.async_copy(
                    ct_hbm.at[idx_all.at[pl.ds(idx_off(s + 2), CHUNK)]],
                    rows[k2], gsem[k2])

            if k == 0:
                pltpu.sync_copy(pe_sh.at[pl.ds(pc * CHUNK, CHUNK)], pe_v)

            pltpu.make_async_copy(ct_hbm.at[idx_all.at[pl.ds(0, CHUNK)]],
                                  rx, gs).wait()

            @pl.when(s >= OBUF)
            def _():
                pltpu.make_async_copy(ob, out_hbm.at[pl.ds(base, CHUNK)],
                                      ws).wait()

            @plsc.parallel_loop(0, PAIRS, step=1, unroll=1)
            def _(c):
                ci = pl.multiple_of(c * LANES, LANES)
                co = pl.multiple_of(c * 32, 32)
                for r in range(CHUNK):
                    uc = rx[r, pl.ds(ci, LANES)]
                    up = pe_v[r, pl.ds(ci, LANES)]
                    hi = jnp.int32(-65536)
                    a = (lax.bitcast_convert_type(uc << 16, jnp.float32)
                         + lax.bitcast_convert_type(up << 16, jnp.float32))
                    b = (lax.bitcast_convert_type(uc & hi, jnp.float32)
                         + lax.bitcast_convert_type(up & hi, jnp.float32))
                    ob[r, pl.ds(co, LANES)] = a
                    ob[r, pl.ds(co + LANES, LANES)] = b

            pltpu.async_copy(ob, out_hbm.at[pl.ds(base, CHUNK)], ws)
        return carry

    lax.fori_loop(0, nsteps // NBUF, outer, 0, unroll=False)
    for k in ((nsteps - 2) % OBUF, (nsteps - 1) % OBUF):
        pltpu.make_async_copy(obufs[k], out_hbm.at[pl.ds(w0, CHUNK)],
                              wsem[k]).wait()


@jax.jit
def _embed(gidx, ct, pe):
    n_rows = gidx.shape[0]
    mesh = plsc.VectorSubcoreMesh(core_axis_name="c", subcore_axis_name="s")
    f = functools.partial(
        pl.kernel,
        out_type=jax.ShapeDtypeStruct((n_rows, D), jnp.float32),
        mesh=mesh,
        scratch_types=[
            pltpu.VMEM((n_rows // NW,), jnp.int32),
            pltpu.VMEM((CHUNK, D // 2), jnp.int32),
            [pltpu.VMEM((CHUNK, D // 2), jnp.int32)] * NBUF,
            [pltpu.VMEM((CHUNK, D), jnp.float32)] * OBUF,
            pltpu.VMEM_SHARED((512, D // 2), jnp.int32),
            [pltpu.SemaphoreType.DMA] * NBUF,
            [pltpu.SemaphoreType.DMA] * OBUF,
        ],
    )(_sc_body)
    return f(gidx, ct, pe)


def kernel(tokens, actions, tok_embed0, tok_embed1, tok_embed2, action_embed,
           level_embed, pos_embed):
    B, T, _ = tokens.shape
    num_codes = tok_embed0.shape[0]
    ct = jnp.concatenate(
        [
            tok_embed0 + level_embed[0],
            tok_embed1 + level_embed[1],
            tok_embed2 + level_embed[2],
            action_embed + level_embed[3],
        ],
        axis=0,
    )
    perm = jnp.asarray(_PERM)
    ct = ct[:, perm].astype(jnp.bfloat16)
    ct = lax.bitcast_convert_type(ct.reshape(ct.shape[0], D // 2, 2), jnp.int32)
    pe = pos_embed[: T * 4][:, perm].astype(jnp.bfloat16)
    pe = lax.bitcast_convert_type(pe.reshape(T * 4, D // 2, 2), jnp.int32)
    gidx = jnp.stack(
        [
            tokens[..., 0],
            tokens[..., 1] + num_codes,
            tokens[..., 2] + 2 * num_codes,
            actions + 3 * num_codes,
        ],
        axis=-1,
    ).reshape(-1)
    out = _embed(gidx, ct, pe)
    return out.reshape(B, T * 4, D)


# speedup vs baseline: 1.0589x; 1.0023x over previous
import functools

import jax
import jax.numpy as jnp
import numpy as np
from jax import lax
from jax.experimental import pallas as pl
from jax.experimental.pallas import tpu as pltpu
from jax.experimental.pallas import tpu_sc as plsc

D = 768
LANES = 16
PAIRS = D // (2 * LANES)
NW = 32
NS = 16
B_PER_W = 4
CHUNK = 16
NBUF = 4
OBUF = 2

_PERM = np.arange(D).reshape(PAIRS, 2, LANES).transpose(0, 2, 1).reshape(-1)


def _sc_body(gidx_hbm, ct_hbm, pe_hbm, out_hbm,
             idx_all, pe_v, rows, obufs, pe_sh, gsem, wsem):
    n_rows = out_hbm.shape[0]
    p_per_b = pe_hbm.shape[0]
    rows_per_w = n_rows // NW
    pcs = p_per_b // CHUNK
    nsteps = pcs * B_PER_W
    sid = lax.axis_index("s")
    wid = sid * 2 + lax.axis_index("c")
    w0 = wid * rows_per_w

    pps = p_per_b // NS
    for j in range(pps // CHUNK):
        r0 = sid * pps + j * CHUNK
        pltpu.sync_copy(pe_hbm.at[pl.ds(r0, CHUNK)], rows[0])
        pltpu.sync_copy(rows[0], pe_sh.at[pl.ds(r0, CHUNK)])
    plsc.subcore_barrier()

    def idx_off(s):
        pc = s // B_PER_W
        bi = lax.rem(s, B_PER_W)
        return bi * p_per_b + pc * CHUNK

    pltpu.sync_copy(gidx_hbm.at[pl.ds(w0, rows_per_w)], idx_all)
    pltpu.async_copy(ct_hbm.at[idx_all.at[pl.ds(idx_off(0), CHUNK)]], rows[0], gsem[0])
    pltpu.async_copy(ct_hbm.at[idx_all.at[pl.ds(idx_off(1), CHUNK)]], rows[1], gsem[1])

    def outer(i, carry):
        for k in range(NBUF):
            s = i * NBUF + k
            pc = s // B_PER_W
            base = w0 + lax.rem(s, B_PER_W) * p_per_b + pc * CHUNK
            rx, gs = rows[k], gsem[k]
            ob, ws = obufs[k % OBUF], wsem[k % OBUF]
            k2 = (k + 2) % NBUF
            @pl.when(s + 2 < nsteps)
            def _():
                pltpu.async_copy(
                    ct_hbm.at[idx_all.at[pl.ds(idx_off(s + 2), CHUNK)]],
                    rows[k2], gsem[k2])

            if k == 0:
                pltpu.sync_copy(pe_sh.at[pl.ds(pc * CHUNK, CHUNK)], pe_v)

            pltpu.make_async_copy(ct_hbm.at[idx_all.at[pl.ds(0, CHUNK)]],
                                  rx, gs).wait()

            @pl.when(s >= OBUF)
            def _():
                pltpu.make_async_copy(ob, out_hbm.at[pl.ds(base, CHUNK)],
                                      ws).wait()

            @plsc.parallel_loop(0, PAIRS, step=1, unroll=1)
            def _(c):
                ci = pl.multiple_of(c * LANES, LANES)
                co = pl.multiple_of(c * 32, 32)
                for r in range(CHUNK):
                    uc = rx[r, pl.ds(ci, LANES)]
                    up = pe_v[r, pl.ds(ci, LANES)]
                    hi = jnp.int32(-65536)
                    a = (lax.bitcast_convert_type(uc << 16, jnp.float32)
                         + lax.bitcast_convert_type(up << 16, jnp.float32))
                    b = (lax.bitcast_convert_type(uc & hi, jnp.float32)
                         + lax.bitcast_convert_type(up & hi, jnp.float32))
                    ob[r, pl.ds(co, LANES)] = a
                    ob[r, pl.ds(co + LANES, LANES)] = b

            pltpu.async_copy(ob, out_hbm.at[pl.ds(base, CHUNK)], ws)
        return carry

    lax.fori_loop(0, nsteps // NBUF, outer, 0, unroll=False)
    for k in ((nsteps - 2) % OBUF, (nsteps - 1) % OBUF):
        pltpu.make_async_copy(obufs[k], out_hbm.at[pl.ds(w0, CHUNK)],
                              wsem[k]).wait()


@jax.jit
def _embed(gidx, ct, pe):
    n_rows = gidx.shape[0]
    mesh = plsc.VectorSubcoreMesh(core_axis_name="c", subcore_axis_name="s")
    f = functools.partial(
        pl.kernel,
        out_type=jax.ShapeDtypeStruct((n_rows, D), jnp.float32),
        mesh=mesh,
        scratch_types=[
            pltpu.VMEM((n_rows // NW,), jnp.int32),
            pltpu.VMEM((CHUNK, D // 2), jnp.int32),
            [pltpu.VMEM((CHUNK, D // 2), jnp.int32)] * NBUF,
            [pltpu.VMEM((CHUNK, D), jnp.float32)] * OBUF,
            pltpu.VMEM_SHARED((512, D // 2), jnp.int32),
            [pltpu.SemaphoreType.DMA] * NBUF,
            [pltpu.SemaphoreType.DMA] * OBUF,
        ],
    )(_sc_body)
    return f(gidx, ct, pe)


def kernel(tokens, actions, tok_embed0, tok_embed1, tok_embed2, action_embed,
           level_embed, pos_embed):
    B, T, _ = tokens.shape
    num_codes = tok_embed0.shape[0]
    ct = jnp.concatenate(
        [
            tok_embed0 + level_embed[0],
            tok_embed1 + level_embed[1],
            tok_embed2 + level_embed[2],
            action_embed + level_embed[3],
        ],
        axis=0,
    )
    perm = jnp.asarray(_PERM)
    ct = ct[:, perm].astype(jnp.bfloat16)
    ct = lax.bitcast_convert_type(ct.reshape(ct.shape[0], D // 2, 2), jnp.int32)
    pe = pos_embed[: T * 4][:, perm].astype(jnp.bfloat16)
    pe = lax.bitcast_convert_type(pe.reshape(T * 4, D // 2, 2), jnp.int32)
    gidx = jnp.stack(
        [
            tokens[..., 0],
            tokens[..., 1] + num_codes,
            tokens[..., 2] + 2 * num_codes,
            actions + 3 * num_codes,
        ],
        axis=-1,
    ).reshape(-1)
    out = _embed(gidx, ct, pe)
    return out.reshape(B, T * 4, D)
